# baseline (speedup 1.0000x reference)
import functools
import math

import jax
import jax.numpy as jnp
from jax import lax
from jax.experimental import pallas as pl
from jax.experimental.pallas import tpu as pltpu
from jax.experimental.pallas import tpu_sc as plsc

_V = 32000
_PAD = 0
_SMOOTHING = 0.1
_CONF = 1.0 - _SMOOTHING
_EPS = _SMOOTHING / (_V - 1)
_C = (_V - 1) * _EPS * math.log(_EPS) + _CONF * math.log(_CONF)

_NC = 2
_NS = 16
_L = 16
_NW = _NC * _NS
_NBUF = 8
_G = 16


def _sc_body(x_hbm, tgt_hbm, sums_hbm, cnts_hbm, drows_hbm,
             tgt_v, vec_v, cvec_v, dout_v,
             buf0, buf1, buf2, buf3, buf4, buf5, buf6, buf7,
             sem0, sem1, sem2, sem3, sem4, sem5, sem6, sem7):
    chunk = tgt_v.shape[0]
    wid = lax.axis_index("s") * _NC + lax.axis_index("c")
    base = wid * chunk
    pltpu.sync_copy(tgt_hbm.at[pl.ds(base, chunk)], tgt_v)
    iota = lax.iota(jnp.int32, _L)

    bufs = (buf0, buf1, buf2, buf3, buf4, buf5, buf6, buf7)
    sems = (sem0, sem1, sem2, sem3, sem4, sem5, sem6, sem7)
    _H = _V // 2

    def _issue(h, b):
        pltpu.async_copy(
            x_hbm.at[pl.ds(base + h // 2, 1), pl.ds((h % 2) * _H, _H)],
            bufs[b], sems[b])

    for b in range(_NBUF):
        _issue(b, b)
    vec_v[...] = jnp.zeros((_L,), jnp.float32)

    def _half_sum(buf, init):
        def inner(k, a):
            b = k * (16 * _L)
            return tuple(a[j] + buf[0, pl.ds(b + _L * j, _L)]
                         for j in range(16))

        accs = list(lax.fori_loop(0, _H // (16 * _L), inner, init))
        b0 = (_H // (16 * _L)) * 16 * _L
        for j in range(8):
            accs[j] = accs[j] + buf[0, pl.ds(b0 + _L * j, _L)]
        return tuple(accs)

    def _extract(buf, t_r, lo):
        off = jnp.minimum(jnp.maximum((t_r >> 4) * _L - lo, 0), _H - _L)
        seg = buf[0, pl.ds(off, _L)]
        sel = jnp.where(iota == (t_r & (_L - 1)), seg, jnp.float32(0.0))
        inh = (t_r >= lo) & (t_r < lo + _H)
        validf = jnp.where(inh & (t_r != _PAD), jnp.float32(1.0),
                           jnp.float32(0.0))
        return sel * validf

    zero = jnp.zeros((_L,), jnp.float32)

    @pl.loop(0, chunk // _G)
    def _group(g):
        r0 = g * _G
        t16 = tgt_v[pl.ds(r0, _G)]
        for k in range(_G):
            r = r0 + k
            t_r = t16[k]
            hit = zero
            accs = (zero,) * 16
            for h in range(2):
                hr = 2 * k + h
                b = hr % _NBUF
                pltpu.make_async_copy(
                    x_hbm.at[pl.ds(base + r, 1), pl.ds(h * _H, _H)],
                    bufs[b], sems[b]).wait()
                accs = _half_sum(bufs[b], accs)
                hit = hit + _extract(bufs[b], t_r, h * _H)

                @pl.when(2 * r + h + _NBUF < 2 * chunk)
                def _():
                    _issue(2 * r + h + _NBUF, b)

            w = accs[0]
            for j in range(1, 16):
                w = w + accs[j]
            dout_v[pl.ds(r * _L, _L)] = w
            vec_v[...] = vec_v[...] + hit

    cnt = jnp.zeros((_L,), jnp.int32)
    for j in range(chunk // _L):
        t16 = tgt_v[pl.ds(j * _L, _L)]
        cnt = cnt + jnp.where(t16 != _PAD, 1, 0)
    cvec_v[...] = cnt
    pltpu.sync_copy(vec_v, sums_hbm.at[wid])
    pltpu.sync_copy(cvec_v, cnts_hbm.at[wid])
    pltpu.sync_copy(dout_v, drows_hbm.at[wid])


def _sc_pass(x, tgt):
    n = tgt.shape[0]
    chunk = n // _NW
    mesh = plsc.VectorSubcoreMesh(core_axis_name="c", subcore_axis_name="s")
    run = functools.partial(
        pl.kernel,
        out_type=(
            jax.ShapeDtypeStruct((_NW, _L), jnp.float32),
            jax.ShapeDtypeStruct((_NW, _L), jnp.int32),
            jax.ShapeDtypeStruct((_NW, chunk * _L), jnp.float32),
        ),
        mesh=mesh,
        scratch_types=(
            pltpu.VMEM((chunk,), jnp.int32),
            pltpu.VMEM((_L,), jnp.float32),
            pltpu.VMEM((_L,), jnp.int32),
            pltpu.VMEM((chunk * _L,), jnp.float32),
            *([pltpu.VMEM((1, _V // 2), jnp.float32)] * 8),
            *([pltpu.SemaphoreType.DMA] * 8),
        ),
    )(_sc_body)
    return run(x, tgt)


def _comb_body(sums_ref, cnts_ref, drows_ref, trep_ref, out_ref):
    s_t = jnp.sum(sums_ref[...])
    nv = jnp.sum(cnts_ref[...]).astype(jnp.float32)
    dmask = (trep_ref[...] != _PAD).astype(jnp.float32)
    s_all = jnp.sum(drows_ref[...] * dmask)
    out_ref[0, 0] = (nv * jnp.float32(_C)
                     - jnp.float32(_EPS) * s_all
                     - jnp.float32(_CONF - _EPS) * s_t) / nv


def _combine(sums, cnts, drows, trep):
    return pl.pallas_call(
        _comb_body,
        in_specs=[
            pl.BlockSpec(),
            pl.BlockSpec(),
            pl.BlockSpec(),
            pl.BlockSpec(),
        ],
        out_specs=pl.BlockSpec(memory_space=pltpu.SMEM),
        out_shape=jax.ShapeDtypeStruct((1, 1), jnp.float32),
    )(sums, cnts, drows, trep)


def kernel(x, target):
    n, v = x.shape
    target = target.astype(jnp.int32)
    sums, cnts, drows = _sc_pass(x, target)
    trep = jnp.repeat(target, _L).reshape(drows.shape)
    out = _combine(sums, cnts, drows, trep)
    return out[0, 0]


# speedup vs baseline: 1.0204x; 1.0204x over previous
import functools
import math

import jax
import jax.numpy as jnp
from jax import lax
from jax.experimental import pallas as pl
from jax.experimental.pallas import tpu as pltpu
from jax.experimental.pallas import tpu_sc as plsc

_V = 32000
_PAD = 0
_SMOOTHING = 0.1
_CONF = 1.0 - _SMOOTHING
_EPS = _SMOOTHING / (_V - 1)
_C = (_V - 1) * _EPS * math.log(_EPS) + _CONF * math.log(_CONF)

_NC = 2
_NS = 16
_L = 16
_NW = _NC * _NS
_NBUF = 4
_G = 16


def _sc_body(x_hbm, tgt_hbm, sums_hbm, cnts_hbm, drows_hbm,
             tgt_v, vec_v, cvec_v, dout_v, buf0, buf1, buf2, buf3,
             sem0, sem1, sem2, sem3):
    chunk = tgt_v.shape[0]
    wid = lax.axis_index("s") * _NC + lax.axis_index("c")
    base = wid * chunk
    pltpu.sync_copy(tgt_hbm.at[pl.ds(base, chunk)], tgt_v)
    iota = lax.iota(jnp.int32, _L)

    bufs = (buf0, buf1, buf2, buf3)
    sems = (sem0, sem1, sem2, sem3)
    for b in range(_NBUF):
        pltpu.async_copy(x_hbm.at[pl.ds(base + b, 1)], bufs[b], sems[b])
    vec_v[...] = jnp.zeros((_L,), jnp.float32)

    def _row_sum(buf):
        zero = jnp.zeros((_L,), jnp.float32)

        def inner(k, a):
            b = k * (16 * _L)
            return tuple(a[j] + buf[0, pl.ds(b + _L * j, _L)]
                         for j in range(16))

        accs = lax.fori_loop(0, _V // (16 * _L), inner, (zero,) * 16)
        w = accs[0]
        for j in range(1, 16):
            w = w + accs[j]
        return w

    @pl.loop(0, chunk // _G)
    def _group(g):
        r0 = g * _G
        t16 = tgt_v[pl.ds(r0, _G)]
        for k in range(_G):
            r = r0 + k
            buf = bufs[k % _NBUF]
            sem = sems[k % _NBUF]
            pltpu.make_async_copy(
                x_hbm.at[pl.ds(base + r, 1)], buf, sem).wait()
            dout_v[pl.ds(r * _L, _L)] = _row_sum(buf)
            t_r = t16[k]
            seg = buf[0, pl.ds((t_r >> 4) * _L, _L)]
            sel = jnp.where(iota == (t_r & (_L - 1)), seg, jnp.float32(0.0))
            validf = jnp.where(t_r != _PAD, jnp.float32(1.0),
                               jnp.float32(0.0))
            vec_v[...] = vec_v[...] + sel * validf

            @pl.when(r + _NBUF < chunk)
            def _():
                pltpu.async_copy(
                    x_hbm.at[pl.ds(base + r + _NBUF, 1)], buf, sem)

    cnt = jnp.zeros((_L,), jnp.int32)
    for j in range(chunk // _L):
        t16 = tgt_v[pl.ds(j * _L, _L)]
        cnt = cnt + jnp.where(t16 != _PAD, 1, 0)
    cvec_v[...] = cnt
    pltpu.sync_copy(vec_v, sums_hbm.at[wid])
    pltpu.sync_copy(cvec_v, cnts_hbm.at[wid])
    pltpu.sync_copy(dout_v, drows_hbm.at[wid])


def _sc_pass(x, tgt):
    n = tgt.shape[0]
    chunk = n // _NW
    mesh = plsc.VectorSubcoreMesh(core_axis_name="c", subcore_axis_name="s")
    run = functools.partial(
        pl.kernel,
        out_type=(
            jax.ShapeDtypeStruct((_NW, _L), jnp.float32),
            jax.ShapeDtypeStruct((_NW, _L), jnp.int32),
            jax.ShapeDtypeStruct((_NW, chunk * _L), jnp.float32),
        ),
        mesh=mesh,
        scratch_types=(
            pltpu.VMEM((chunk,), jnp.int32),
            pltpu.VMEM((_L,), jnp.float32),
            pltpu.VMEM((_L,), jnp.int32),
            pltpu.VMEM((chunk * _L,), jnp.float32),
            pltpu.VMEM((1, _V), jnp.float32),
            pltpu.VMEM((1, _V), jnp.float32),
            pltpu.VMEM((1, _V), jnp.float32),
            pltpu.VMEM((1, _V), jnp.float32),
            pltpu.SemaphoreType.DMA,
            pltpu.SemaphoreType.DMA,
            pltpu.SemaphoreType.DMA,
            pltpu.SemaphoreType.DMA,
        ),
    )(_sc_body)
    return run(x, tgt)


def _comb_body(sums_ref, cnts_ref, drows_ref, trep_ref, out_ref):
    s_t = jnp.sum(sums_ref[...])
    nv = jnp.sum(cnts_ref[...]).astype(jnp.float32)
    dmask = (trep_ref[...] != _PAD).astype(jnp.float32)
    s_all = jnp.sum(drows_ref[...] * dmask)
    out_ref[0, 0] = (nv * jnp.float32(_C)
                     - jnp.float32(_EPS) * s_all
                     - jnp.float32(_CONF - _EPS) * s_t) / nv


def _combine(sums, cnts, drows, trep):
    return pl.pallas_call(
        _comb_body,
        in_specs=[
            pl.BlockSpec(),
            pl.BlockSpec(),
            pl.BlockSpec(),
            pl.BlockSpec(),
        ],
        out_specs=pl.BlockSpec(memory_space=pltpu.SMEM),
        out_shape=jax.ShapeDtypeStruct((1, 1), jnp.float32),
    )(sums, cnts, drows, trep)


def kernel(x, target):
    n, v = x.shape
    target = target.astype(jnp.int32)
    sums, cnts, drows = _sc_pass(x, target)
    trep = jnp.repeat(target, _L).reshape(drows.shape)
    out = _combine(sums, cnts, drows, trep)
    return out[0, 0]
